# Initial kernel scaffold; baseline (speedup 1.0000x reference)
import jax
import jax.numpy as jnp
from jax.experimental import pallas as pl


def kernel(element_idxs, indices, distances, eps, sigma):
    raise NotImplementedError("write your pallas kernel here")


# speedup vs baseline: 318.2567x; 318.2567x over previous
import dataclasses
import functools

import jax
import jax.numpy as jnp
from jax import lax
from jax.experimental import pallas as pl
from jax.experimental.pallas import tpu as pltpu
from jax.experimental.pallas import tpu_sc as plsc

CUTOFF = 5.2
NMOL = 100
NATOM = 500
NELEM = 4

NC = 2
NS = 16
NW = NC * NS
LANES = 16
BINS_W = 128

_COS_C = (
    0.999999992289847,
    -0.4999999177095917,
    0.04166652433757106,
    -0.0013887970265660749,
    2.4773420813417784e-05,
    -2.711333772340869e-07,
    1.736899605040332e-09,
)


def _sc_pair_kernel(n_pairs, n_atoms_total, chunk):
    n_per_w = n_pairs // NW
    n_chunks = n_per_w // chunk
    mesh = plsc.VectorSubcoreMesh(
        core_axis_name="c", subcore_axis_name="s",
        num_cores=NC, num_subcores=NS)
    inv_cut = float(jnp.pi) / CUTOFF

    cp = pltpu.CompilerParams()
    if "needs_layout_passes" in pltpu.CompilerParams.__dataclass_fields__:
        cp = dataclasses.replace(cp, needs_layout_passes=False)

    @functools.partial(
        pl.kernel,
        mesh=mesh,
        compiler_params=cp,
        out_type=jax.ShapeDtypeStruct((NW, BINS_W), jnp.float32),
        scratch_types=[
            pltpu.VMEM((n_atoms_total,), jnp.int32),
            pltpu.VMEM((NELEM * NELEM,), jnp.float32),
            pltpu.VMEM((NELEM * NELEM,), jnp.float32),
            pltpu.VMEM((LANES, BINS_W), jnp.float32),
            pltpu.VMEM((BINS_W,), jnp.float32),
            pltpu.VMEM((chunk,), jnp.int32),
            pltpu.VMEM((chunk,), jnp.int32),
            pltpu.VMEM((chunk,), jnp.float32),
            pltpu.SemaphoreType.DMA,
        ],
    )
    def k(elem_hbm, i0_hbm, i1_hbm, d_hbm, a_hbm, b_hbm, out_hbm,
          elem_v, a_v, b_v, bins_v, acc_v, i0_v, i1_v, d_v, sem):
        wid = lax.axis_index("s") * NC + lax.axis_index("c")
        base = wid * n_per_w

        pltpu.sync_copy(elem_hbm, elem_v)
        pltpu.sync_copy(a_hbm, a_v)
        pltpu.sync_copy(b_hbm, b_v)

        zeros = jnp.zeros((LANES,), jnp.float32)

        @pl.loop(0, LANES)
        def _zero_rows(r):
            @pl.loop(0, BINS_W, step=LANES)
            def _zero_cols(cc):
                bins_v[r, pl.ds(cc, LANES)] = zeros

        rows = lax.iota(jnp.int32, LANES)

        @pl.loop(0, n_chunks)
        def _chunk(ci):
            off = base + ci * chunk
            pltpu.sync_copy(i0_hbm.at[pl.ds(off, chunk)], i0_v)
            pltpu.sync_copy(i1_hbm.at[pl.ds(off, chunk)], i1_v)
            pltpu.sync_copy(d_hbm.at[pl.ds(off, chunk)], d_v)

            @pl.loop(0, chunk, step=LANES)
            def _vec(j):
                sl = pl.ds(j, LANES)
                vi0 = i0_v[sl]
                vi1 = i1_v[sl]
                e0 = plsc.load_gather(elem_v, [vi0])
                e1 = plsc.load_gather(elem_v, [vi1])
                c = e0 * NELEM + e1
                av = plsc.load_gather(a_v, [c])
                bv = plsc.load_gather(b_v, [c])
                d = d_v[sl]
                inv = 1.0 / d
                inv2 = inv * inv
                inv6 = inv2 * inv2 * inv2
                x = d * inv_cut
                u = x * x
                cosx = jnp.float32(_COS_C[6])
                for cf in (_COS_C[5], _COS_C[4], _COS_C[3],
                           _COS_C[2], _COS_C[1], _COS_C[0]):
                    cosx = cosx * u + jnp.float32(cf)
                fc = 0.5 * cosx + 0.5
                t = inv6 * fc
                e = t * (av * inv6 + bv)
                mf = (vi0.astype(jnp.float32) + 0.5) * (1.0 / NATOM)
                m = mf.astype(jnp.int32)
                plsc.addupdate_scatter(bins_v, [rows, m], e)

        for cc in range(0, BINS_W, LANES):
            sl = pl.ds(cc, LANES)
            s = bins_v[0, sl]
            for r in range(1, LANES):
                s = s + bins_v[r, sl]
            acc_v[sl] = s
        pltpu.sync_copy(acc_v, out_hbm.at[wid])

    return k


def _tc_reduce(x_ref, o_ref):
    o_ref[...] = jnp.sum(x_ref[...], axis=0, keepdims=True)


def kernel(element_idxs, indices, distances, eps, sigma):
    n_mols, n_atoms = element_idxs.shape
    n_pairs = distances.shape[0]
    flat_elem = element_idxs.reshape(-1)
    sig2 = sigma * sigma
    sig6 = sig2 * sig2 * sig2
    a = (4.0 * eps * sig6 * sig6).reshape(-1)
    b = (-4.0 * eps * sig6).reshape(-1)
    sc = _sc_pair_kernel(n_pairs, n_mols * n_atoms, 2000)
    partials = sc(flat_elem, indices[0], indices[1], distances, a, b)
    reduced = pl.pallas_call(
        _tc_reduce,
        out_shape=jax.ShapeDtypeStruct((1, BINS_W), jnp.float32),
    )(partials)
    return reduced[0, :n_mols]


# speedup vs baseline: 597.5756x; 1.8777x over previous
import dataclasses
import functools

import jax
import jax.numpy as jnp
from jax import lax
from jax.experimental import pallas as pl
from jax.experimental.pallas import tpu as pltpu
from jax.experimental.pallas import tpu_sc as plsc

CUTOFF = 5.2
NMOL = 100
NATOM = 500
NELEM = 4

NC = 2
NS = 16
NW = NC * NS
LANES = 16
BINS_W = 128

_COS_C = (
    0.999999992289847,
    -0.4999999177095917,
    0.04166652433757106,
    -0.0013887970265660749,
    2.4773420813417784e-05,
    -2.711333772340869e-07,
    1.736899605040332e-09,
)


def _sc_pair_kernel(n_pairs, n_atoms_total, chunk, unroll):
    n_per_w = n_pairs // NW
    n_chunks = n_per_w // chunk
    mesh = plsc.VectorSubcoreMesh(
        core_axis_name="c", subcore_axis_name="s",
        num_cores=NC, num_subcores=NS)
    inv_cut = float(jnp.pi) / CUTOFF

    cp = pltpu.CompilerParams()
    if "needs_layout_passes" in pltpu.CompilerParams.__dataclass_fields__:
        cp = dataclasses.replace(cp, needs_layout_passes=False)

    @functools.partial(
        pl.kernel,
        mesh=mesh,
        compiler_params=cp,
        out_type=jax.ShapeDtypeStruct((NW, BINS_W), jnp.float32),
        scratch_types=[
            pltpu.VMEM((n_atoms_total,), jnp.int32),
            pltpu.VMEM((NELEM * NELEM,), jnp.float32),
            pltpu.VMEM((NELEM * NELEM,), jnp.float32),
            pltpu.VMEM((LANES, BINS_W), jnp.float32),
            pltpu.VMEM((BINS_W,), jnp.float32),
            pltpu.VMEM((chunk,), jnp.int32),
            pltpu.VMEM((chunk,), jnp.int32),
            pltpu.VMEM((chunk,), jnp.int32),
            pltpu.VMEM((chunk,), jnp.int32),
            pltpu.VMEM((chunk,), jnp.float32),
            pltpu.VMEM((chunk,), jnp.float32),
            pltpu.SemaphoreType.DMA,
            pltpu.SemaphoreType.DMA,
        ],
    )
    def k(packed_hbm, i0_hbm, i1_hbm, d_hbm, a_hbm, b_hbm, out_hbm,
          packed_v, a_v, b_v, bins_v, acc_v,
          i0_a, i0_b, i1_a, i1_b, d_a, d_b, sem0, sem1):
        i0_bufs = (i0_a, i0_b)
        i1_bufs = (i1_a, i1_b)
        d_bufs = (d_a, d_b)
        wid = lax.axis_index("s") * NC + lax.axis_index("c")
        base = wid * n_per_w
        sems = (sem0, sem1)

        tbl = pltpu.async_copy(packed_hbm, packed_v, sem0)
        pltpu.sync_copy(a_hbm, a_v)
        pltpu.sync_copy(b_hbm, b_v)

        zeros = jnp.zeros((LANES,), jnp.float32)

        @pl.loop(0, LANES)
        def _zero_rows(r):
            @pl.loop(0, BINS_W, step=LANES)
            def _zero_cols(cc):
                bins_v[r, pl.ds(cc, LANES)] = zeros

        rows = lax.iota(jnp.int32, LANES)

        def start3(ci, slot):
            off = base + ci * chunk
            s = sems[slot]
            return [
                pltpu.async_copy(i0_hbm.at[pl.ds(off, chunk)],
                                 i0_bufs[slot], s),
                pltpu.async_copy(i1_hbm.at[pl.ds(off, chunk)],
                                 i1_bufs[slot], s),
                pltpu.async_copy(d_hbm.at[pl.ds(off, chunk)],
                                 d_bufs[slot], s),
            ]

        def compute(slot):
            @plsc.parallel_loop(0, chunk, LANES, unroll=unroll)
            def _vec(j):
                sl = pl.ds(j, LANES)
                vi0 = i0_bufs[slot][sl]
                vi1 = i1_bufs[slot][sl]
                p0 = plsc.load_gather(packed_v, [vi0])
                p1 = plsc.load_gather(packed_v, [vi1])
                c = (p0 & 15) + ((p1 & 15) >> 2)
                m = p0 >> 10
                av = plsc.load_gather(a_v, [c])
                bv = plsc.load_gather(b_v, [c])
                d = d_bufs[slot][sl]
                inv = 1.0 / d
                inv2 = inv * inv
                inv6 = inv2 * inv2 * inv2
                x = d * inv_cut
                u = x * x
                cosx = jnp.float32(_COS_C[6])
                for cf in (_COS_C[5], _COS_C[4], _COS_C[3],
                           _COS_C[2], _COS_C[1], _COS_C[0]):
                    cosx = cosx * u + jnp.float32(cf)
                fc = 0.5 * cosx + 0.5
                t = inv6 * fc
                e = t * (av * inv6 + bv)
                plsc.addupdate_scatter(bins_v, [rows, m], e)

        hs = start3(0, 0)
        tbl.wait()
        for ci in range(n_chunks):
            cur = ci % 2
            for h in hs:
                h.wait()
            if ci + 1 < n_chunks:
                hs = start3(ci + 1, 1 - cur)
            compute(cur)

        for cc in range(0, BINS_W, LANES):
            sl = pl.ds(cc, LANES)
            s = bins_v[0, sl]
            for r in range(1, LANES):
                s = s + bins_v[r, sl]
            acc_v[sl] = s
        pltpu.sync_copy(acc_v, out_hbm.at[wid])

    return k


def _tc_pack(x_ref, o_ref):
    n_mols, n_atoms = x_ref.shape
    mol = lax.broadcasted_iota(jnp.int32, (n_mols, n_atoms), 0)
    o_ref[...] = mol * 1024 + x_ref[...] * 4


def _tc_reduce(x_ref, o_ref):
    o_ref[...] = jnp.sum(x_ref[...], axis=0, keepdims=True)


def kernel(element_idxs, indices, distances, eps, sigma):
    n_mols, n_atoms = element_idxs.shape
    n_pairs = distances.shape[0]
    packed = pl.pallas_call(
        _tc_pack,
        out_shape=jax.ShapeDtypeStruct((n_mols, n_atoms), jnp.int32),
    )(element_idxs).reshape(-1)
    sig2 = sigma * sigma
    sig6 = sig2 * sig2 * sig2
    a = (4.0 * eps * sig6 * sig6).reshape(-1)
    b = (-4.0 * eps * sig6).reshape(-1)
    sc = _sc_pair_kernel(n_pairs, n_mols * n_atoms, 10000, 4)
    partials = sc(packed, indices[0], indices[1], distances, a, b)
    reduced = pl.pallas_call(
        _tc_reduce,
        out_shape=jax.ShapeDtypeStruct((1, BINS_W), jnp.float32),
    )(partials)
    return reduced[0, :n_mols]


# speedup vs baseline: 950.7845x; 1.5911x over previous
import dataclasses
import functools

import jax
import jax.numpy as jnp
from jax import lax
from jax.experimental import pallas as pl
from jax.experimental.pallas import tpu as pltpu
from jax.experimental.pallas import tpu_sc as plsc

CUTOFF = 5.2
NMOL = 100
NATOM = 500
NELEM = 4

NC = 2
NS = 16
NW = NC * NS
LANES = 16
BINS_W = 128

_COS_C = (
    0.999999992289847,
    -0.4999999177095917,
    0.04166652433757106,
    -0.0013887970265660749,
    2.4773420813417784e-05,
    -2.711333772340869e-07,
    1.736899605040332e-09,
)


def _sc_pair_kernel(n_pairs, n_atoms_total, chunk, unroll):
    n_per_w = n_pairs // NW
    n_chunks = n_per_w // chunk
    mesh = plsc.VectorSubcoreMesh(
        core_axis_name="c", subcore_axis_name="s",
        num_cores=NC, num_subcores=NS)
    inv_cut = float(jnp.pi) / CUTOFF

    cp = pltpu.CompilerParams()
    if "needs_layout_passes" in pltpu.CompilerParams.__dataclass_fields__:
        cp = dataclasses.replace(cp, needs_layout_passes=False)

    @functools.partial(
        pl.kernel,
        mesh=mesh,
        compiler_params=cp,
        out_type=jax.ShapeDtypeStruct((NW, BINS_W), jnp.float32),
        scratch_types=[
            pltpu.VMEM((n_atoms_total,), jnp.int32),
            pltpu.VMEM((NELEM * NELEM,), jnp.float32),
            pltpu.VMEM((NELEM * NELEM,), jnp.float32),
            pltpu.VMEM((LANES, BINS_W), jnp.float32),
            pltpu.VMEM((BINS_W,), jnp.float32),
            pltpu.VMEM((chunk,), jnp.int32),
            pltpu.VMEM((chunk,), jnp.int32),
            pltpu.VMEM((chunk,), jnp.int32),
            pltpu.VMEM((chunk,), jnp.int32),
            pltpu.VMEM((chunk,), jnp.float32),
            pltpu.VMEM((chunk,), jnp.float32),
            pltpu.SemaphoreType.DMA,
            pltpu.SemaphoreType.DMA,
        ],
    )
    def k(packed_hbm, i0_hbm, i1_hbm, d_hbm, a_hbm, b_hbm, out_hbm,
          packed_v, a_v, b_v, bins_v, acc_v,
          i0_a, i0_b, i1_a, i1_b, d_a, d_b, sem0, sem1):
        i0_bufs = (i0_a, i0_b)
        i1_bufs = (i1_a, i1_b)
        d_bufs = (d_a, d_b)
        wid = lax.axis_index("s") * NC + lax.axis_index("c")
        base = wid * n_per_w
        sems = (sem0, sem1)

        tbl = pltpu.async_copy(packed_hbm, packed_v, sem0)
        pltpu.sync_copy(a_hbm, a_v)
        pltpu.sync_copy(b_hbm, b_v)

        zeros = jnp.zeros((LANES,), jnp.float32)

        @pl.loop(0, LANES)
        def _zero_rows(r):
            @pl.loop(0, BINS_W, step=LANES)
            def _zero_cols(cc):
                bins_v[r, pl.ds(cc, LANES)] = zeros

        rows = lax.iota(jnp.int32, LANES)

        def start3(ci, slot):
            off = base + ci * chunk
            s = sems[slot]
            return [
                pltpu.async_copy(i0_hbm.at[pl.ds(off, chunk)],
                                 i0_bufs[slot], s),
                pltpu.async_copy(i1_hbm.at[pl.ds(off, chunk)],
                                 i1_bufs[slot], s),
                pltpu.async_copy(d_hbm.at[pl.ds(off, chunk)],
                                 d_bufs[slot], s),
            ]

        def compute(slot):
            @plsc.parallel_loop(0, chunk, LANES, unroll=unroll)
            def _vec(j):
                sl = pl.ds(j, LANES)
                vi0 = i0_bufs[slot][sl]
                vi1 = i1_bufs[slot][sl]
                p0 = plsc.load_gather(packed_v, [vi0])
                p1 = plsc.load_gather(packed_v, [vi1])
                c = (p0 & 15) + ((p1 & 15) >> 2)
                m = p0 >> 10
                av = plsc.load_gather(a_v, [c])
                bv = plsc.load_gather(b_v, [c])
                d = d_bufs[slot][sl]
                inv = 1.0 / d
                inv2 = inv * inv
                inv6 = inv2 * inv2 * inv2
                x = d * inv_cut
                u = x * x
                cosx = jnp.float32(_COS_C[6])
                for cf in (_COS_C[5], _COS_C[4], _COS_C[3],
                           _COS_C[2], _COS_C[1], _COS_C[0]):
                    cosx = cosx * u + jnp.float32(cf)
                fc = 0.5 * cosx + 0.5
                t = inv6 * fc
                e = t * (av * inv6 + bv)
                plsc.addupdate_scatter(bins_v, [rows, m], e)

        hs = start3(0, 0)
        tbl.wait()
        for ci in range(n_chunks):
            cur = ci % 2
            for h in hs:
                h.wait()
            if ci + 1 < n_chunks:
                hs = start3(ci + 1, 1 - cur)
            compute(cur)

        for cc in range(0, BINS_W, LANES):
            sl = pl.ds(cc, LANES)
            s = bins_v[0, sl]
            for r in range(1, LANES):
                s = s + bins_v[r, sl]
            acc_v[sl] = s
        pltpu.sync_copy(acc_v, out_hbm.at[wid])

    return k


def _tc_split(blk, x_ref, o0_ref, o1_ref):
    i = pl.program_id(0)
    o0_ref[pl.ds(i * blk, blk)] = x_ref[0, :]
    o1_ref[pl.ds(i * blk, blk)] = x_ref[1, :]


def _tc_pack(x_ref, o_ref):
    n_mols, n_atoms = x_ref.shape
    mol = lax.broadcasted_iota(jnp.int32, (n_mols, n_atoms), 0)
    o_ref[...] = mol * 1024 + x_ref[...] * 4


def _tc_reduce(x_ref, o_ref):
    o_ref[...] = jnp.sum(x_ref[...], axis=0, keepdims=True)


def kernel(element_idxs, indices, distances, eps, sigma):
    n_mols, n_atoms = element_idxs.shape
    n_pairs = distances.shape[0]
    packed = pl.pallas_call(
        _tc_pack,
        out_shape=jax.ShapeDtypeStruct((n_mols, n_atoms), jnp.int32),
    )(element_idxs).reshape(-1)
    sig2 = sigma * sigma
    sig6 = sig2 * sig2 * sig2
    a = (4.0 * eps * sig6 * sig6).reshape(-1)
    b = (-4.0 * eps * sig6).reshape(-1)
    blk = 64000
    i0, i1 = pl.pallas_call(
        functools.partial(_tc_split, blk),
        grid=(n_pairs // blk,),
        in_specs=[pl.BlockSpec((2, blk), lambda i: (0, i))],
        out_specs=[pl.BlockSpec((n_pairs,), lambda i: (0,)),
                   pl.BlockSpec((n_pairs,), lambda i: (0,))],
        out_shape=[jax.ShapeDtypeStruct((n_pairs,), jnp.int32),
                   jax.ShapeDtypeStruct((n_pairs,), jnp.int32)],
    )(indices)
    sc = _sc_pair_kernel(n_pairs, n_mols * n_atoms, 10000, 8)
    partials = sc(packed, i0, i1, distances, a, b)
    reduced = pl.pallas_call(
        _tc_reduce,
        out_shape=jax.ShapeDtypeStruct((1, BINS_W), jnp.float32),
    )(partials)
    return reduced[0, :n_mols]


# speedup vs baseline: 1087.9991x; 1.1443x over previous
import dataclasses
import functools

import jax
import jax.numpy as jnp
from jax import lax
from jax.experimental import pallas as pl
from jax.experimental.pallas import tpu as pltpu
from jax.experimental.pallas import tpu_sc as plsc

CUTOFF = 5.2
NMOL = 100
NATOM = 500
NELEM = 4

NC = 2
NS = 16
NW = NC * NS
LANES = 16
TILE = 128
BINS_W = 128

_COS_C = (
    0.999999992289847,
    -0.4999999177095917,
    0.04166652433757106,
    -0.0013887970265660749,
    2.4773420813417784e-05,
    -2.711333772340869e-07,
    1.736899605040332e-09,
)


def _sc_pair_kernel(n_pairs, n_atoms_total, ct, unroll):
    n_tiles = n_pairs // TILE
    n_chunks = n_tiles // ct
    n_rounds = -(-n_chunks // NW)
    full_w = n_chunks - (n_rounds - 1) * NW
    mesh = plsc.VectorSubcoreMesh(
        core_axis_name="c", subcore_axis_name="s",
        num_cores=NC, num_subcores=NS)
    inv_cut = float(jnp.pi) / CUTOFF

    cp = pltpu.CompilerParams()
    if "needs_layout_passes" in pltpu.CompilerParams.__dataclass_fields__:
        cp = dataclasses.replace(cp, needs_layout_passes=False)

    @functools.partial(
        pl.kernel,
        mesh=mesh,
        compiler_params=cp,
        out_type=jax.ShapeDtypeStruct((NW, BINS_W), jnp.float32),
        scratch_types=[
            pltpu.VMEM((n_atoms_total,), jnp.int32),
            pltpu.VMEM((NELEM * NELEM,), jnp.float32),
            pltpu.VMEM((NELEM * NELEM,), jnp.float32),
            pltpu.VMEM((LANES, BINS_W), jnp.float32),
            pltpu.VMEM((BINS_W,), jnp.float32),
            pltpu.VMEM((ct, 2, TILE), jnp.int32),
            pltpu.VMEM((ct, 2, TILE), jnp.int32),
            pltpu.VMEM((ct * TILE,), jnp.float32),
            pltpu.VMEM((ct * TILE,), jnp.float32),
            pltpu.SemaphoreType.DMA,
            pltpu.SemaphoreType.DMA,
        ],
    )
    def k(packed_hbm, it_hbm, d_hbm, a_hbm, b_hbm, out_hbm,
          packed_v, a_v, b_v, bins_v, acc_v, iv_a, iv_b, d_a, d_b,
          sem0, sem1):
        iv_bufs = (iv_a, iv_b)
        d_bufs = (d_a, d_b)
        sems = (sem0, sem1)
        wid = lax.axis_index("s") * NC + lax.axis_index("c")
        in_last = wid < full_w

        def chunk_idx(r):
            ci = wid + r * NW
            if r == n_rounds - 1:
                ci = lax.select(in_last, ci, wid)
            return ci

        tbl = pltpu.async_copy(packed_hbm, packed_v, sem0)
        pltpu.sync_copy(a_hbm, a_v)
        pltpu.sync_copy(b_hbm, b_v)

        zeros = jnp.zeros((LANES,), jnp.float32)

        @pl.loop(0, LANES)
        def _zero_rows(r):
            @pl.loop(0, BINS_W, step=LANES)
            def _zero_cols(cc):
                bins_v[r, pl.ds(cc, LANES)] = zeros

        rows = lax.iota(jnp.int32, LANES)

        def start2(ci, slot):
            s = sems[slot]
            return [
                pltpu.async_copy(it_hbm.at[pl.ds(ci * ct, ct)],
                                 iv_bufs[slot], s),
                pltpu.async_copy(d_hbm.at[pl.ds(ci * ct * TILE, ct * TILE)],
                                 d_bufs[slot], s),
            ]

        def compute(slot):
            iv = iv_bufs[slot]
            dv = d_bufs[slot]

            @plsc.parallel_loop(0, ct, 1, unroll=unroll)
            def _vec(g):
                db = g * TILE
                for t in range(TILE // LANES):
                    sl = pl.ds(t * LANES, LANES)
                    vi0 = iv[g, 0, sl]
                    vi1 = iv[g, 1, sl]
                    p0 = plsc.load_gather(packed_v, [vi0])
                    p1 = plsc.load_gather(packed_v, [vi1])
                    c = (p0 & 15) + ((p1 & 15) >> 2)
                    m = p0 >> 10
                    av = plsc.load_gather(a_v, [c])
                    bv = plsc.load_gather(b_v, [c])
                    d = dv[pl.ds(db + t * LANES, LANES)]
                    inv = 1.0 / d
                    inv2 = inv * inv
                    inv6 = inv2 * inv2 * inv2
                    x = d * inv_cut
                    u = x * x
                    cosx = jnp.float32(_COS_C[6])
                    for cf in (_COS_C[5], _COS_C[4], _COS_C[3],
                               _COS_C[2], _COS_C[1], _COS_C[0]):
                        cosx = cosx * u + jnp.float32(cf)
                    fc = 0.5 * cosx + 0.5
                    tt = inv6 * fc
                    e = tt * (av * inv6 + bv)
                    plsc.addupdate_scatter(bins_v, [rows, m], e)

        hs = start2(chunk_idx(0), 0)
        tbl.wait()
        for r in range(n_rounds):
            cur = r % 2
            for h in hs:
                h.wait()
            if r + 1 < n_rounds:
                hs = start2(chunk_idx(r + 1), 1 - cur)
            if r == n_rounds - 1:
                @pl.when(in_last)
                def _last():
                    compute(cur)
            else:
                compute(cur)

        for cc in range(0, BINS_W, LANES):
            sl = pl.ds(cc, LANES)
            s = bins_v[0, sl]
            for r in range(1, LANES):
                s = s + bins_v[r, sl]
            acc_v[sl] = s
        pltpu.sync_copy(acc_v, out_hbm.at[wid])

    return k


def _tc_pack(x_ref, o_ref):
    n_mols, n_atoms = x_ref.shape
    mol = lax.broadcasted_iota(jnp.int32, (n_mols, n_atoms), 0)
    o_ref[...] = mol * 1024 + x_ref[...] * 4


def _tc_reduce(x_ref, o_ref):
    o_ref[...] = jnp.sum(x_ref[...], axis=0, keepdims=True)


def kernel(element_idxs, indices, distances, eps, sigma):
    n_mols, n_atoms = element_idxs.shape
    n_pairs = distances.shape[0]
    packed = pl.pallas_call(
        _tc_pack,
        out_shape=jax.ShapeDtypeStruct((n_mols, n_atoms), jnp.int32),
    )(element_idxs).reshape(-1)
    sig2 = sigma * sigma
    sig6 = sig2 * sig2 * sig2
    a = (4.0 * eps * sig6 * sig6).reshape(-1)
    b = (-4.0 * eps * sig6).reshape(-1)
    it = indices.reshape(2, n_pairs // TILE, TILE).transpose(1, 0, 2)
    sc = _sc_pair_kernel(n_pairs, n_mols * n_atoms, 50, 1)
    partials = sc(packed, it, distances, a, b)
    reduced = pl.pallas_call(
        _tc_reduce,
        out_shape=jax.ShapeDtypeStruct((1, BINS_W), jnp.float32),
    )(partials)
    return reduced[0, :n_mols]
